# baseline (speedup 1.0000x reference)
import functools

import jax
import jax.numpy as jnp
from jax import lax
from jax.experimental import pallas as pl
from jax.experimental.pallas import tpu as pltpu
from jax.experimental.pallas import tpu_sc as plsc

BATCH = 256
CH = 128
NC, NS_SUB = 2, 16
NQ = 4
NS = 8
W = BATCH // NS
LANES = 16
LEAK = 0.01


def _sc_layer(n_in, n_out, npc):
    mesh = plsc.VectorSubcoreMesh(core_axis_name="c", subcore_axis_name="s")
    acc_n = n_out * W

    @functools.partial(
        pl.kernel,
        out_type=jax.ShapeDtypeStruct((NQ, NS, acc_n), jnp.float32),
        mesh=mesh,
        compiler_params=pltpu.CompilerParams(
            needs_layout_passes=False, use_tc_tiling_on_sc=False
        ),
        scratch_types=[
            pltpu.VMEM((acc_n,), jnp.float32),
            pltpu.VMEM((CH,), jnp.int32),
            pltpu.VMEM((CH,), jnp.int32),
            pltpu.VMEM((CH,), jnp.float32),
            pltpu.VMEM((CH, W), jnp.float32),
            pltpu.SemaphoreType.DMA,
        ],
    )
    def k(xsrc, col, roww, w, out, acc, idxc, idxr, wv, rows, sem):
        cid = lax.axis_index("c")
        sid = lax.axis_index("s")
        wid = sid * NC + cid
        q = wid % NQ
        s = wid // NQ
        iot = lax.iota(jnp.int32, LANES)

        zeros = jnp.zeros((LANES,), jnp.float32)

        def zbody(i, carry):
            for u in range(8):
                acc[pl.ds((i * 8 + u) * LANES, LANES)] = zeros
            return carry

        lax.fori_loop(0, acc_n // (8 * LANES), zbody, 0)

        def chunk_body(c, carry):
            pltpu.sync_copy(col.at[q, c], idxc)
            pltpu.sync_copy(roww.at[q, c], idxr)
            pltpu.sync_copy(w.at[q, c], wv)
            pltpu.async_copy(xsrc.at[s].at[idxc], rows, sem).wait()

            def group(g, gcarry):
                w16 = wv[pl.ds(g * LANES, LANES)]
                r16 = idxr[pl.ds(g * LANES, LANES)]
                for ii in range(LANES):
                    wb = w16[ii]
                    base = lax.broadcast(r16[ii], (LANES,)) + iot
                    e = g * LANES + ii
                    for u in range(W // LANES):
                        v = rows[e, pl.ds(u * LANES, LANES)] * wb
                        plsc.addupdate_scatter(acc, [base + u * LANES], v)
                return gcarry

            lax.fori_loop(0, CH // LANES, group, 0)
            return carry

        lax.fori_loop(0, npc, chunk_body, 0)

        pltpu.sync_copy(acc, out.at[q, s])

    return k


def _finalize(n_out, bs):
    def body(p_ref, b_ref, o_ref):
        y = p_ref[0] + p_ref[1] + p_ref[2] + p_ref[3]
        y = y + b_ref[...][jnp.newaxis, :, :]
        o_ref[...] = jnp.where(y >= 0, y, LEAK * y)

    return pl.pallas_call(
        body,
        grid=(n_out // bs,),
        in_specs=[
            pl.BlockSpec((NQ, NS, bs, W), lambda i: (0, 0, i, 0)),
            pl.BlockSpec((bs, 1), lambda i: (i, 0)),
        ],
        out_specs=pl.BlockSpec((NS, bs, W), lambda i: (0, i, 0)),
        out_shape=jax.ShapeDtypeStruct((NS, n_out, W), jnp.float32),
    )


def _prep_edges(row, col, w, n_in, n_out, npc):
    e_pad = NQ * npc * CH
    pad = e_pad - row.shape[0]
    pad_col = jnp.arange(pad, dtype=jnp.int32) % n_in
    pad_row = jnp.arange(pad, dtype=jnp.int32) % n_out
    col = jnp.concatenate([col.astype(jnp.int32), pad_col]).reshape(NQ, npc, CH)
    roww = jnp.concatenate([row.astype(jnp.int32), pad_row]).reshape(NQ, npc, CH) * W
    w = jnp.concatenate([w, jnp.zeros((pad,), w.dtype)]).reshape(NQ, npc, CH)
    return roww, col, w


def _layer(xsrc, row, col, w, b, n_out):
    n_in = xsrc.shape[1]
    npc = -(-row.shape[0] // (NQ * CH))
    roww, col, w = _prep_edges(row, col, w, n_in, n_out, npc)
    partials = _sc_layer(n_in, n_out, npc)(xsrc, col, roww, w)
    p4 = partials.reshape(NQ, NS, n_out, W)
    bs = n_out // 8
    return _finalize(n_out, bs)(p4, b.reshape(n_out, 1))


def kernel(features, row1, col1, w1, b1, row2, col2, w2, b2, row3, col3, w3, b3):
    hid = b1.shape[0]
    tf = b3.shape[0]
    gene = features.shape[1]
    xsrc = features.reshape(NS, W, gene).transpose(0, 2, 1)
    h = _layer(xsrc, row1, col1, w1, b1, hid)
    h = _layer(h, row2, col2, w2, b2, hid)
    emb = _layer(h, row3, col3, w3, b3, tf)
    return emb.transpose(0, 2, 1).reshape(BATCH, tf)
